# Initial kernel scaffold; baseline (speedup 1.0000x reference)
import jax
import jax.numpy as jnp
from jax.experimental import pallas as pl


def kernel(x, edge_index, W, b):
    raise NotImplementedError("write your pallas kernel here")


# speedup vs baseline: 24.2577x; 24.2577x over previous
import functools

import jax
import jax.numpy as jnp
from jax import lax
from jax.experimental import pallas as pl
from jax.experimental.pallas import tpu as pltpu
from jax.experimental.pallas import tpu_sc as plsc

NP = 10240
DP = 48
CH = 128
NTILES = 32
ROWS_PER_TILE = NP // 16

_mesh = plsc.VectorSubcoreMesh(core_axis_name="c", subcore_axis_name="s")


def _deg_kernel_factory(n_chunks_per_tile, e_flat_per_tile):
  @functools.partial(
      pl.kernel,
      out_type=jax.ShapeDtypeStruct((NTILES, NP), jnp.float32),
      mesh=_mesh,
      scratch_types=[
          pltpu.VMEM((e_flat_per_tile,), jnp.int32),
          pltpu.VMEM((NP,), jnp.float32),
      ],
      compiler_params=pltpu.CompilerParams(needs_layout_passes=False),
  )
  def deg_kernel(dst_hbm, out_hbm, idx_v, deg_v):
    c = lax.axis_index("c")
    s = lax.axis_index("s")
    wid = s * 2 + c

    def zero_body(i, carry):
      deg_v[pl.ds(i * 16, 16)] = jnp.zeros((16,), jnp.float32)
      return carry

    lax.fori_loop(0, NP // 16, zero_body, 0)

    pltpu.sync_copy(dst_hbm.at[pl.ds(wid * e_flat_per_tile, e_flat_per_tile)],
                    idx_v)

    ones = jnp.ones((16,), jnp.float32)

    def body(i, carry):
      idx = idx_v[pl.ds(i * 16, 16)]
      plsc.addupdate_scatter(deg_v, [idx], ones)
      return carry

    lax.fori_loop(0, e_flat_per_tile // 16, body, 0)

    pltpu.sync_copy(deg_v, out_hbm.at[wid])

  return deg_kernel


def _prop_kernel_factory(n_chunks_per_tile):
  @functools.partial(
      pl.kernel,
      out_type=jax.ShapeDtypeStruct((2, NP, DP), jnp.float32),
      mesh=_mesh,
      scratch_types=[
          pltpu.VMEM((n_chunks_per_tile, CH), jnp.int32),
          pltpu.VMEM((n_chunks_per_tile, CH), jnp.int32),
          pltpu.VMEM((CH, DP), jnp.float32),
          pltpu.VMEM_SHARED((NP, DP), jnp.float32),
          pltpu.SemaphoreType.DMA,
      ],
      compiler_params=pltpu.CompilerParams(use_tc_tiling_on_sc=False),
  )
  def prop_kernel(g_hbm, src_hbm, dst_hbm, out_hbm, sidx_v, didx_v, rows_v,
                  acc_sh, sem):
    c = lax.axis_index("c")
    s = lax.axis_index("s")
    wid = s * 2 + c

    def zrow_body(i, carry):
      r = i // (DP // 16)
      k = i % (DP // 16)
      rows_v[r, pl.ds(k * 16, 16)] = jnp.zeros((16,), jnp.float32)
      return carry

    lax.fori_loop(0, CH * (DP // 16), zrow_body, 0)

    def zcopy_body(z, carry):
      pltpu.sync_copy(rows_v, acc_sh.at[pl.ds(s * ROWS_PER_TILE + z * CH, CH)])
      return carry

    lax.fori_loop(0, ROWS_PER_TILE // CH, zcopy_body, 0)

    pltpu.sync_copy(src_hbm.at[pl.ds(wid * n_chunks_per_tile,
                                     n_chunks_per_tile)], sidx_v)
    pltpu.sync_copy(dst_hbm.at[pl.ds(wid * n_chunks_per_tile,
                                     n_chunks_per_tile)], didx_v)

    plsc.subcore_barrier()

    def chunk_body(j, carry):
      pltpu.async_copy(g_hbm.at[sidx_v.at[j]], rows_v, sem).wait()
      pltpu.sync_copy(rows_v, acc_sh.at[didx_v.at[j]], add=True)
      return carry

    lax.fori_loop(0, n_chunks_per_tile, chunk_body, 0)

    plsc.subcore_barrier()

    pltpu.sync_copy(acc_sh.at[pl.ds(s * ROWS_PER_TILE, ROWS_PER_TILE)],
                    out_hbm.at[c, pl.ds(s * ROWS_PER_TILE, ROWS_PER_TILE)])

  return prop_kernel


def _mm_body(cnt_ref, x_ref, w_ref, g_ref, dinv_ref):
  deg = jnp.sum(cnt_ref[...], axis=0) + 1.0
  dinv = lax.rsqrt(deg)
  h = jnp.dot(x_ref[...], w_ref[...], preferred_element_type=jnp.float32)
  g_ref[...] = h * dinv[:, None]
  dinv_ref[...] = dinv[None, :]


def _fin_body(s_ref, g_ref, dinv_ref, b_ref, o_ref):
  t = s_ref[0] + s_ref[1] + g_ref[...]
  o_ref[...] = t * dinv_ref[0][:, None] + b_ref[...]


def kernel(x, edge_index, W, b):
  n = x.shape[0]
  e = edge_index.shape[1]
  d_out = W.shape[1]

  src = edge_index[0].astype(jnp.int32)
  dst = edge_index[1].astype(jnp.int32)
  e_quant = NTILES * CH * 8
  e_pad = ((e + e_quant - 1) // e_quant) * e_quant
  pad_amt = e_pad - e
  src_p = jnp.concatenate([src, jnp.full((pad_amt,), n, jnp.int32)])
  dst_p = jnp.concatenate([dst, jnp.full((pad_amt,), n, jnp.int32)])
  n_chunks = e_pad // CH
  cpt = n_chunks // NTILES
  src2d = src_p.reshape(n_chunks, CH)
  dst2d = dst_p.reshape(n_chunks, CH)

  xp = jnp.pad(x, ((0, NP - n), (0, 0)))
  Wp = jnp.pad(W, ((0, 0), (0, DP - d_out)))
  b2 = jnp.pad(b, (0, DP - d_out))[None, :]

  cnt = _deg_kernel_factory(cpt, cpt * CH)(dst_p)

  blk = 1024
  g, dinv = pl.pallas_call(
      _mm_body,
      grid=(NP // blk,),
      in_specs=[
          pl.BlockSpec((NTILES, blk), lambda i: (0, i)),
          pl.BlockSpec((blk, 128), lambda i: (i, 0)),
          pl.BlockSpec((128, DP), lambda i: (0, 0)),
      ],
      out_specs=[
          pl.BlockSpec((blk, DP), lambda i: (i, 0)),
          pl.BlockSpec((1, blk), lambda i: (0, i)),
      ],
      out_shape=[
          jax.ShapeDtypeStruct((NP, DP), jnp.float32),
          jax.ShapeDtypeStruct((1, NP), jnp.float32),
      ],
  )(cnt, xp, Wp)

  S = _prop_kernel_factory(cpt)(g, src2d, dst2d)

  outp = pl.pallas_call(
      _fin_body,
      grid=(NP // blk,),
      in_specs=[
          pl.BlockSpec((2, blk, DP), lambda i: (0, i, 0)),
          pl.BlockSpec((blk, DP), lambda i: (i, 0)),
          pl.BlockSpec((1, blk), lambda i: (0, i)),
          pl.BlockSpec((1, DP), lambda i: (0, 0)),
      ],
      out_specs=pl.BlockSpec((blk, DP), lambda i: (i, 0)),
      out_shape=jax.ShapeDtypeStruct((NP, DP), jnp.float32),
  )(S, g, dinv, b2)

  return outp[:n, :d_out]


# speedup vs baseline: 27.4958x; 1.1335x over previous
import functools

import jax
import jax.numpy as jnp
from jax import lax
from jax.experimental import pallas as pl
from jax.experimental.pallas import tpu as pltpu
from jax.experimental.pallas import tpu_sc as plsc

NP = 10240
DP = 48
CH = 128
NTILES = 32
ROWS_PER_TILE = NP // 16

_mesh = plsc.VectorSubcoreMesh(core_axis_name="c", subcore_axis_name="s")


def _deg_kernel_factory(n_chunks_per_tile, e_flat_per_tile):
  @functools.partial(
      pl.kernel,
      out_type=jax.ShapeDtypeStruct((NTILES, NP), jnp.float32),
      mesh=_mesh,
      scratch_types=[
          pltpu.VMEM((e_flat_per_tile,), jnp.int32),
          pltpu.VMEM((NP,), jnp.float32),
      ],
      compiler_params=pltpu.CompilerParams(needs_layout_passes=False),
  )
  def deg_kernel(dst_hbm, out_hbm, idx_v, deg_v):
    c = lax.axis_index("c")
    s = lax.axis_index("s")
    wid = s * 2 + c

    def zero_body(i, carry):
      deg_v[pl.ds(i * 16, 16)] = jnp.zeros((16,), jnp.float32)
      return carry

    lax.fori_loop(0, NP // 16, zero_body, 0)

    pltpu.sync_copy(dst_hbm.at[pl.ds(wid * e_flat_per_tile, e_flat_per_tile)],
                    idx_v)

    ones = jnp.ones((16,), jnp.float32)

    def body(i, carry):
      idx = idx_v[pl.ds(i * 16, 16)]
      plsc.addupdate_scatter(deg_v, [idx], ones)
      return carry

    lax.fori_loop(0, e_flat_per_tile // 16, body, 0)

    pltpu.sync_copy(deg_v, out_hbm.at[wid])

  return deg_kernel


NBUF = 4


def _prop_kernel_factory(n_chunks_per_tile):
  n_groups = n_chunks_per_tile // NBUF

  @functools.partial(
      pl.kernel,
      out_type=jax.ShapeDtypeStruct((2, NP, DP), jnp.float32),
      mesh=_mesh,
      scratch_types=[
          pltpu.VMEM((n_chunks_per_tile, CH), jnp.int32),
          pltpu.VMEM((n_chunks_per_tile, CH), jnp.int32),
          pltpu.VMEM((2, NBUF, CH, DP), jnp.float32),
          pltpu.VMEM_SHARED((NP, DP), jnp.float32),
          pltpu.SemaphoreType.DMA,
          pltpu.SemaphoreType.DMA,
      ],
      compiler_params=pltpu.CompilerParams(use_tc_tiling_on_sc=False),
  )
  def prop_kernel(g_hbm, src_hbm, dst_hbm, out_hbm, sidx_v, didx_v, rows_v,
                  acc_sh, gsem, ssem):
    c = lax.axis_index("c")
    s = lax.axis_index("s")
    wid = s * 2 + c

    def zrow_body(i, carry):
      r = i // (DP // 16)
      k = i % (DP // 16)
      rows_v[0, 0, r, pl.ds(k * 16, 16)] = jnp.zeros((16,), jnp.float32)
      return carry

    lax.fori_loop(0, CH * (DP // 16), zrow_body, 0)

    def zcopy_body(z, carry):
      pltpu.sync_copy(rows_v.at[0, 0],
                      acc_sh.at[pl.ds(s * ROWS_PER_TILE + z * CH, CH)])
      return carry

    lax.fori_loop(0, ROWS_PER_TILE // CH, zcopy_body, 0)

    pltpu.sync_copy(src_hbm.at[pl.ds(wid * n_chunks_per_tile,
                                     n_chunks_per_tile)], sidx_v)
    pltpu.sync_copy(dst_hbm.at[pl.ds(wid * n_chunks_per_tile,
                                     n_chunks_per_tile)], didx_v)

    plsc.subcore_barrier()

    def gather_start(g, p, k):
      pltpu.async_copy(g_hbm.at[sidx_v.at[g * NBUF + k]], rows_v.at[p, k],
                       gsem)

    def gather_wait(g, p, k):
      pltpu.make_async_copy(g_hbm.at[sidx_v.at[g * NBUF + k]],
                            rows_v.at[p, k], gsem).wait()

    def scatter_start(g, p, k):
      pltpu.async_copy(rows_v.at[p, k], acc_sh.at[didx_v.at[g * NBUF + k]],
                       ssem, add=True)

    def scatter_wait(g, p, k):
      pltpu.make_async_copy(rows_v.at[p, k],
                            acc_sh.at[didx_v.at[g * NBUF + k]], ssem).wait()

    for k in range(NBUF):
      gather_start(0, 0, k)

    def group_body(g, carry):
      p = lax.rem(g, 2)
      for k in range(NBUF):
        gather_wait(g, p, k)

      @pl.when(g < n_groups - 1)
      def _():
        for k in range(NBUF):
          gather_start(g + 1, 1 - p, k)

      for k in range(NBUF):
        scatter_start(g, p, k)
      for k in range(NBUF):
        scatter_wait(g, p, k)
      return carry

    lax.fori_loop(0, n_groups, group_body, 0)

    plsc.subcore_barrier()

    pltpu.sync_copy(acc_sh.at[pl.ds(s * ROWS_PER_TILE, ROWS_PER_TILE)],
                    out_hbm.at[c, pl.ds(s * ROWS_PER_TILE, ROWS_PER_TILE)])

  return prop_kernel


def _mm_body(cnt_ref, x_ref, w_ref, g_ref, dinv_ref):
  deg = jnp.sum(cnt_ref[...], axis=0) + 1.0
  dinv = lax.rsqrt(deg)
  h = jnp.dot(x_ref[...], w_ref[...], preferred_element_type=jnp.float32)
  g_ref[...] = h * dinv[:, None]
  dinv_ref[...] = dinv[None, :]


def _fin_body(s_ref, g_ref, dinv_ref, b_ref, o_ref):
  t = s_ref[0] + s_ref[1] + g_ref[...]
  o_ref[...] = t * dinv_ref[0][:, None] + b_ref[...]


def kernel(x, edge_index, W, b):
  n = x.shape[0]
  e = edge_index.shape[1]
  d_out = W.shape[1]

  src = edge_index[0].astype(jnp.int32)
  dst = edge_index[1].astype(jnp.int32)
  e_quant = NTILES * CH * 8
  e_pad = ((e + e_quant - 1) // e_quant) * e_quant
  pad_amt = e_pad - e
  src_p = jnp.concatenate([src, jnp.full((pad_amt,), n, jnp.int32)])
  dst_p = jnp.concatenate([dst, jnp.full((pad_amt,), n, jnp.int32)])
  n_chunks = e_pad // CH
  cpt = n_chunks // NTILES
  src2d = src_p.reshape(n_chunks, CH)
  dst2d = dst_p.reshape(n_chunks, CH)

  xp = jnp.pad(x, ((0, NP - n), (0, 0)))
  Wp = jnp.pad(W, ((0, 0), (0, DP - d_out)))
  b2 = jnp.pad(b, (0, DP - d_out))[None, :]

  cnt = _deg_kernel_factory(cpt, cpt * CH)(dst_p)

  blk = 1024
  g, dinv = pl.pallas_call(
      _mm_body,
      grid=(NP // blk,),
      in_specs=[
          pl.BlockSpec((NTILES, blk), lambda i: (0, i)),
          pl.BlockSpec((blk, 128), lambda i: (i, 0)),
          pl.BlockSpec((128, DP), lambda i: (0, 0)),
      ],
      out_specs=[
          pl.BlockSpec((blk, DP), lambda i: (i, 0)),
          pl.BlockSpec((1, blk), lambda i: (0, i)),
      ],
      out_shape=[
          jax.ShapeDtypeStruct((NP, DP), jnp.float32),
          jax.ShapeDtypeStruct((1, NP), jnp.float32),
      ],
  )(cnt, xp, Wp)

  S = _prop_kernel_factory(cpt)(g, src2d, dst2d)

  outp = pl.pallas_call(
      _fin_body,
      grid=(NP // blk,),
      in_specs=[
          pl.BlockSpec((2, blk, DP), lambda i: (0, i, 0)),
          pl.BlockSpec((blk, DP), lambda i: (i, 0)),
          pl.BlockSpec((1, blk), lambda i: (0, i)),
          pl.BlockSpec((1, DP), lambda i: (0, 0)),
      ],
      out_specs=pl.BlockSpec((blk, DP), lambda i: (i, 0)),
      out_shape=jax.ShapeDtypeStruct((NP, DP), jnp.float32),
  )(S, g, dinv, b2)

  return outp[:n, :d_out]


# speedup vs baseline: 52.1664x; 1.8972x over previous
import functools

import jax
import jax.numpy as jnp
from jax import lax
from jax.experimental import pallas as pl
from jax.experimental.pallas import tpu as pltpu
from jax.experimental.pallas import tpu_sc as plsc

NP = 10240
DP = 48
CH = 128
NTILES = 32
ROWS_PER_TILE = NP // 16

_mesh = plsc.VectorSubcoreMesh(core_axis_name="c", subcore_axis_name="s")


def _deg_kernel_factory(n_chunks_per_tile, e_flat_per_tile):
  @functools.partial(
      pl.kernel,
      out_type=jax.ShapeDtypeStruct((NTILES, NP), jnp.float32),
      mesh=_mesh,
      scratch_types=[
          pltpu.VMEM((e_flat_per_tile,), jnp.int32),
          pltpu.VMEM((NP,), jnp.float32),
      ],
      compiler_params=pltpu.CompilerParams(needs_layout_passes=False),
  )
  def deg_kernel(dst_hbm, out_hbm, idx_v, deg_v):
    c = lax.axis_index("c")
    s = lax.axis_index("s")
    wid = s * 2 + c

    def zero_body(i, carry):
      deg_v[pl.ds(i * 16, 16)] = jnp.zeros((16,), jnp.float32)
      return carry

    lax.fori_loop(0, NP // 16, zero_body, 0)

    pltpu.sync_copy(dst_hbm.at[pl.ds(wid * e_flat_per_tile, e_flat_per_tile)],
                    idx_v)

    ones = jnp.ones((16,), jnp.float32)

    def body(i, carry):
      idx = idx_v[pl.ds(i * 16, 16)]
      plsc.addupdate_scatter(deg_v, [idx], ones)
      return carry

    lax.fori_loop(0, e_flat_per_tile // 16, body, 0)

    pltpu.sync_copy(deg_v, out_hbm.at[wid])

  return deg_kernel


NBUF = 4


def _prop_kernel_factory(n_chunks_per_tile):
  n_groups = n_chunks_per_tile // NBUF

  @functools.partial(
      pl.kernel,
      out_type=jax.ShapeDtypeStruct((2, NP, DP), jnp.float32),
      mesh=_mesh,
      scratch_types=[
          pltpu.VMEM((n_chunks_per_tile, CH), jnp.int32),
          pltpu.VMEM((n_chunks_per_tile, CH), jnp.int32),
          pltpu.VMEM((2, NBUF, CH, DP), jnp.float32),
          pltpu.VMEM_SHARED((NP, DP), jnp.float32),
          pltpu.VMEM_SHARED((NP, DP), jnp.float32),
          pltpu.SemaphoreType.DMA,
          pltpu.SemaphoreType.DMA,
      ],
      compiler_params=pltpu.CompilerParams(use_tc_tiling_on_sc=False),
  )
  def prop_kernel(g_hbm, src_hbm, dst_hbm, out_hbm, sidx_v, didx_v, rows_v,
                  acc_sh, gtab_sh, gsem, ssem):
    c = lax.axis_index("c")
    s = lax.axis_index("s")
    wid = s * 2 + c

    def zrow_body(i, carry):
      r = i // (DP // 16)
      k = i % (DP // 16)
      rows_v[0, 0, r, pl.ds(k * 16, 16)] = jnp.zeros((16,), jnp.float32)
      return carry

    lax.fori_loop(0, CH * (DP // 16), zrow_body, 0)

    def zcopy_body(z, carry):
      pltpu.sync_copy(rows_v.at[0, 0],
                      acc_sh.at[pl.ds(s * ROWS_PER_TILE + z * CH, CH)])
      return carry

    lax.fori_loop(0, ROWS_PER_TILE // CH, zcopy_body, 0)

    pltpu.sync_copy(src_hbm.at[pl.ds(wid * n_chunks_per_tile,
                                     n_chunks_per_tile)], sidx_v)
    pltpu.sync_copy(dst_hbm.at[pl.ds(wid * n_chunks_per_tile,
                                     n_chunks_per_tile)], didx_v)
    pltpu.sync_copy(g_hbm.at[pl.ds(s * ROWS_PER_TILE, ROWS_PER_TILE)],
                    gtab_sh.at[pl.ds(s * ROWS_PER_TILE, ROWS_PER_TILE)])

    plsc.subcore_barrier()

    def gather_start(g, p, k):
      pltpu.async_copy(gtab_sh.at[sidx_v.at[g * NBUF + k]], rows_v.at[p, k],
                       gsem)

    def gather_wait(g, p, k):
      pltpu.make_async_copy(gtab_sh.at[sidx_v.at[g * NBUF + k]],
                            rows_v.at[p, k], gsem).wait()

    def scatter_start(g, p, k):
      pltpu.async_copy(rows_v.at[p, k], acc_sh.at[didx_v.at[g * NBUF + k]],
                       ssem, add=True)

    def scatter_wait(g, p, k):
      pltpu.make_async_copy(rows_v.at[p, k],
                            acc_sh.at[didx_v.at[g * NBUF + k]], ssem).wait()

    for k in range(NBUF):
      gather_start(0, 0, k)

    def group_body(g, carry):
      p = lax.rem(g, 2)
      for k in range(NBUF):
        gather_wait(g, p, k)

      @pl.when(g < n_groups - 1)
      def _():
        for k in range(NBUF):
          gather_start(g + 1, 1 - p, k)

      for k in range(NBUF):
        scatter_start(g, p, k)
      for k in range(NBUF):
        scatter_wait(g, p, k)
      return carry

    lax.fori_loop(0, n_groups, group_body, 0)

    plsc.subcore_barrier()

    pltpu.sync_copy(acc_sh.at[pl.ds(s * ROWS_PER_TILE, ROWS_PER_TILE)],
                    out_hbm.at[c, pl.ds(s * ROWS_PER_TILE, ROWS_PER_TILE)])

  return prop_kernel


def _mm_body(cnt_ref, x_ref, w_ref, g_ref, dinv_ref):
  deg = jnp.sum(cnt_ref[...], axis=0) + 1.0
  dinv = lax.rsqrt(deg)
  h = jnp.dot(x_ref[...], w_ref[...], preferred_element_type=jnp.float32)
  g_ref[...] = h * dinv[:, None]
  dinv_ref[...] = dinv[None, :]


def _fin_body(s_ref, g_ref, dinv_ref, b_ref, o_ref):
  t = s_ref[0] + s_ref[1] + g_ref[...]
  o_ref[...] = t * dinv_ref[0][:, None] + b_ref[...]


def kernel(x, edge_index, W, b):
  n = x.shape[0]
  e = edge_index.shape[1]
  d_out = W.shape[1]

  src = edge_index[0].astype(jnp.int32)
  dst = edge_index[1].astype(jnp.int32)
  e_quant = NTILES * CH * 8
  e_pad = ((e + e_quant - 1) // e_quant) * e_quant
  pad_amt = e_pad - e
  src_p = jnp.concatenate([src, jnp.full((pad_amt,), n, jnp.int32)])
  dst_p = jnp.concatenate([dst, jnp.full((pad_amt,), n, jnp.int32)])
  n_chunks = e_pad // CH
  cpt = n_chunks // NTILES
  src2d = src_p.reshape(n_chunks, CH)
  dst2d = dst_p.reshape(n_chunks, CH)

  xp = jnp.pad(x, ((0, NP - n), (0, 0)))
  Wp = jnp.pad(W, ((0, 0), (0, DP - d_out)))
  b2 = jnp.pad(b, (0, DP - d_out))[None, :]

  cnt = _deg_kernel_factory(cpt, cpt * CH)(dst_p)

  blk = 1024
  g, dinv = pl.pallas_call(
      _mm_body,
      grid=(NP // blk,),
      in_specs=[
          pl.BlockSpec((NTILES, blk), lambda i: (0, i)),
          pl.BlockSpec((blk, 128), lambda i: (i, 0)),
          pl.BlockSpec((128, DP), lambda i: (0, 0)),
      ],
      out_specs=[
          pl.BlockSpec((blk, DP), lambda i: (i, 0)),
          pl.BlockSpec((1, blk), lambda i: (0, i)),
      ],
      out_shape=[
          jax.ShapeDtypeStruct((NP, DP), jnp.float32),
          jax.ShapeDtypeStruct((1, NP), jnp.float32),
      ],
  )(cnt, xp, Wp)

  S = _prop_kernel_factory(cpt)(g, src2d, dst2d)

  outp = pl.pallas_call(
      _fin_body,
      grid=(NP // blk,),
      in_specs=[
          pl.BlockSpec((2, blk, DP), lambda i: (0, i, 0)),
          pl.BlockSpec((blk, DP), lambda i: (i, 0)),
          pl.BlockSpec((1, blk), lambda i: (0, i)),
          pl.BlockSpec((1, DP), lambda i: (0, 0)),
      ],
      out_specs=pl.BlockSpec((blk, DP), lambda i: (i, 0)),
      out_shape=jax.ShapeDtypeStruct((NP, DP), jnp.float32),
  )(S, g, dinv, b2)

  return outp[:n, :d_out]


# speedup vs baseline: 55.0609x; 1.0555x over previous
import functools

import jax
import jax.numpy as jnp
from jax import lax
from jax.experimental import pallas as pl
from jax.experimental.pallas import tpu as pltpu
from jax.experimental.pallas import tpu_sc as plsc

NP = 10240
DP = 40
CH = 128
NTILES = 32
ROWS_PER_TILE = NP // 16

_mesh = plsc.VectorSubcoreMesh(core_axis_name="c", subcore_axis_name="s")


def _deg_kernel_factory(n_chunks_per_tile, e_flat_per_tile):
  @functools.partial(
      pl.kernel,
      out_type=jax.ShapeDtypeStruct((NTILES, NP), jnp.float32),
      mesh=_mesh,
      scratch_types=[
          pltpu.VMEM((e_flat_per_tile,), jnp.int32),
          pltpu.VMEM((NP,), jnp.float32),
      ],
      compiler_params=pltpu.CompilerParams(needs_layout_passes=False),
  )
  def deg_kernel(dst_hbm, out_hbm, idx_v, deg_v):
    c = lax.axis_index("c")
    s = lax.axis_index("s")
    wid = s * 2 + c

    def zero_body(i, carry):
      for u in range(8):
        deg_v[pl.ds(i * 128 + u * 16, 16)] = jnp.zeros((16,), jnp.float32)
      return carry

    lax.fori_loop(0, NP // 128, zero_body, 0)

    pltpu.sync_copy(dst_hbm.at[pl.ds(wid * e_flat_per_tile, e_flat_per_tile)],
                    idx_v)

    ones = jnp.ones((16,), jnp.float32)

    def body(i, carry):
      for u in range(8):
        idx = idx_v[pl.ds(i * 128 + u * 16, 16)]
        plsc.addupdate_scatter(deg_v, [idx], ones)
      return carry

    lax.fori_loop(0, e_flat_per_tile // 128, body, 0)

    pltpu.sync_copy(deg_v, out_hbm.at[wid])

  return deg_kernel


NBUF = 4


def _prop_kernel_factory(n_chunks_per_tile):
  n_groups = n_chunks_per_tile // NBUF

  @functools.partial(
      pl.kernel,
      out_type=jax.ShapeDtypeStruct((2, NP, DP), jnp.float32),
      mesh=_mesh,
      scratch_types=[
          pltpu.VMEM((n_chunks_per_tile, CH), jnp.int32),
          pltpu.VMEM((n_chunks_per_tile, CH), jnp.int32),
          pltpu.VMEM((2, NBUF, CH, DP), jnp.float32),
          pltpu.VMEM_SHARED((NP, DP), jnp.float32),
          pltpu.VMEM_SHARED((NP, DP), jnp.float32),
          pltpu.SemaphoreType.DMA,
          pltpu.SemaphoreType.DMA,
      ],
      compiler_params=pltpu.CompilerParams(use_tc_tiling_on_sc=False),
  )
  def prop_kernel(g_hbm, src_hbm, dst_hbm, zeros_hbm, out_hbm, sidx_v, didx_v,
                  rows_v, acc_sh, gtab_sh, gsem, ssem):
    c = lax.axis_index("c")
    s = lax.axis_index("s")
    wid = s * 2 + c

    pltpu.sync_copy(zeros_hbm.at[pl.ds(s * ROWS_PER_TILE, ROWS_PER_TILE)],
                    acc_sh.at[pl.ds(s * ROWS_PER_TILE, ROWS_PER_TILE)])

    pltpu.sync_copy(src_hbm.at[pl.ds(wid * n_chunks_per_tile,
                                     n_chunks_per_tile)], sidx_v)
    pltpu.sync_copy(dst_hbm.at[pl.ds(wid * n_chunks_per_tile,
                                     n_chunks_per_tile)], didx_v)
    pltpu.sync_copy(g_hbm.at[pl.ds(s * ROWS_PER_TILE, ROWS_PER_TILE)],
                    gtab_sh.at[pl.ds(s * ROWS_PER_TILE, ROWS_PER_TILE)])

    plsc.subcore_barrier()

    def gather_start(g, p, k):
      pltpu.async_copy(gtab_sh.at[sidx_v.at[g * NBUF + k]], rows_v.at[p, k],
                       gsem)

    def gather_wait(g, p, k):
      pltpu.make_async_copy(gtab_sh.at[sidx_v.at[g * NBUF + k]],
                            rows_v.at[p, k], gsem).wait()

    def scatter_start(g, p, k):
      pltpu.async_copy(rows_v.at[p, k], acc_sh.at[didx_v.at[g * NBUF + k]],
                       ssem, add=True)

    def scatter_wait(g, p, k):
      pltpu.make_async_copy(rows_v.at[p, k],
                            acc_sh.at[didx_v.at[g * NBUF + k]], ssem).wait()

    for k in range(NBUF):
      gather_start(0, 0, k)

    def group_body(g, carry):
      p = lax.rem(g, 2)
      for k in range(NBUF):
        gather_wait(g, p, k)

      @pl.when(g < n_groups - 1)
      def _():
        for k in range(NBUF):
          gather_start(g + 1, 1 - p, k)

      for k in range(NBUF):
        scatter_start(g, p, k)
      for k in range(NBUF):
        scatter_wait(g, p, k)
      return carry

    lax.fori_loop(0, n_groups, group_body, 0)

    plsc.subcore_barrier()

    pltpu.sync_copy(acc_sh.at[pl.ds(s * ROWS_PER_TILE, ROWS_PER_TILE)],
                    out_hbm.at[c, pl.ds(s * ROWS_PER_TILE, ROWS_PER_TILE)])

  return prop_kernel


def _mm_body(cnt_ref, x_ref, w_ref, g_ref, dinv_ref):
  deg = jnp.sum(cnt_ref[...], axis=0) + 1.0
  dinv = lax.rsqrt(deg)
  h = jnp.dot(x_ref[...], w_ref[...], preferred_element_type=jnp.float32)
  g_ref[...] = h * dinv[:, None]
  dinv_ref[...] = dinv[None, :]


def _fin_body(s_ref, g_ref, dinv_ref, b_ref, o_ref):
  t = s_ref[0] + s_ref[1] + g_ref[...]
  o_ref[...] = t * dinv_ref[0][:, None] + b_ref[...]


def kernel(x, edge_index, W, b):
  n = x.shape[0]
  e = edge_index.shape[1]
  d_out = W.shape[1]

  src = edge_index[0].astype(jnp.int32)
  dst = edge_index[1].astype(jnp.int32)
  e_quant = NTILES * CH * 8
  e_pad = ((e + e_quant - 1) // e_quant) * e_quant
  pad_amt = e_pad - e
  src_p = jnp.concatenate([src, jnp.full((pad_amt,), n, jnp.int32)])
  dst_p = jnp.concatenate([dst, jnp.full((pad_amt,), n, jnp.int32)])
  n_chunks = e_pad // CH
  cpt = n_chunks // NTILES
  src2d = src_p.reshape(n_chunks, CH)
  dst2d = dst_p.reshape(n_chunks, CH)

  xp = jnp.pad(x, ((0, NP - n), (0, 0)))
  Wp = jnp.pad(W, ((0, 0), (0, DP - d_out)))
  b2 = jnp.pad(b, (0, DP - d_out))[None, :]

  cnt = _deg_kernel_factory(cpt, cpt * CH)(dst_p)

  blk = 1024
  g, dinv = pl.pallas_call(
      _mm_body,
      grid=(NP // blk,),
      in_specs=[
          pl.BlockSpec((NTILES, blk), lambda i: (0, i)),
          pl.BlockSpec((blk, 128), lambda i: (i, 0)),
          pl.BlockSpec((128, DP), lambda i: (0, 0)),
      ],
      out_specs=[
          pl.BlockSpec((blk, DP), lambda i: (i, 0)),
          pl.BlockSpec((1, blk), lambda i: (0, i)),
      ],
      out_shape=[
          jax.ShapeDtypeStruct((NP, DP), jnp.float32),
          jax.ShapeDtypeStruct((1, NP), jnp.float32),
      ],
  )(cnt, xp, Wp)

  zeros = jnp.zeros((NP, DP), jnp.float32)
  S = _prop_kernel_factory(cpt)(g, src2d, dst2d, zeros)

  outp = pl.pallas_call(
      _fin_body,
      grid=(NP // blk,),
      in_specs=[
          pl.BlockSpec((2, blk, DP), lambda i: (0, i, 0)),
          pl.BlockSpec((blk, DP), lambda i: (i, 0)),
          pl.BlockSpec((1, blk), lambda i: (0, i)),
          pl.BlockSpec((1, DP), lambda i: (0, 0)),
      ],
      out_specs=pl.BlockSpec((blk, DP), lambda i: (i, 0)),
      out_shape=jax.ShapeDtypeStruct((NP, DP), jnp.float32),
  )(S, g, dinv, b2)

  return outp[:n, :d_out]


# speedup vs baseline: 61.5366x; 1.1176x over previous
import functools

import jax
import jax.numpy as jnp
from jax import lax
from jax.experimental import pallas as pl
from jax.experimental.pallas import tpu as pltpu
from jax.experimental.pallas import tpu_sc as plsc

NP = 10240
DP = 40
NTILES = 32
ROWS_PER_TILE = NP // 16

_mesh = plsc.VectorSubcoreMesh(core_axis_name="c", subcore_axis_name="s")


def _deg_kernel_factory(ept):
  @functools.partial(
      pl.kernel,
      out_type=jax.ShapeDtypeStruct((NTILES, NP), jnp.float32),
      mesh=_mesh,
      scratch_types=[
          pltpu.VMEM((ept,), jnp.int32),
          pltpu.VMEM((NP,), jnp.float32),
      ],
      compiler_params=pltpu.CompilerParams(needs_layout_passes=False,
                                           use_tc_tiling_on_sc=False),
  )
  def deg_kernel(edge_hbm, out_hbm, idx_v, deg_v):
    c = lax.axis_index("c")
    s = lax.axis_index("s")
    wid = s * 2 + c

    def zero_body(i, carry):
      for u in range(8):
        deg_v[pl.ds(i * 128 + u * 16, 16)] = jnp.zeros((16,), jnp.float32)
      return carry

    lax.fori_loop(0, NP // 128, zero_body, 0)

    pltpu.sync_copy(edge_hbm.at[1, pl.ds(wid * ept, ept)], idx_v)

    ones = jnp.ones((16,), jnp.float32)

    def body(i, carry):
      for u in range(8):
        idx = idx_v[pl.ds(i * 128 + u * 16, 16)]
        plsc.addupdate_scatter(deg_v, [idx], ones)
      return carry

    lax.fori_loop(0, ept // 128, body, 0)
    for u in range((ept % 128) // 16):
      idx = idx_v[pl.ds((ept // 128) * 128 + u * 16, 16)]
      plsc.addupdate_scatter(deg_v, [idx], ones)

    pltpu.sync_copy(deg_v, out_hbm.at[wid])

  return deg_kernel


NBUF = 2
CH = 200


def _prop_kernel_factory(ept):
  n_chunks = ept // CH
  n_groups = n_chunks // NBUF

  @functools.partial(
      pl.kernel,
      out_type=jax.ShapeDtypeStruct((2, NP, DP), jnp.float32),
      mesh=_mesh,
      scratch_types=[
          pltpu.VMEM((ept,), jnp.int32),
          pltpu.VMEM((ept,), jnp.int32),
          pltpu.VMEM((2, NBUF, CH, DP), jnp.float32),
          pltpu.VMEM_SHARED((NP, DP), jnp.float32),
          pltpu.VMEM_SHARED((NP, DP), jnp.float32),
          pltpu.SemaphoreType.DMA,
          pltpu.SemaphoreType.DMA,
      ],
      compiler_params=pltpu.CompilerParams(use_tc_tiling_on_sc=False),
  )
  def prop_kernel(g_hbm, edge_hbm, zeros_hbm, out_hbm, sidx_v, didx_v,
                  rows_v, acc_sh, gtab_sh, gsem, ssem):
    c = lax.axis_index("c")
    s = lax.axis_index("s")
    wid = s * 2 + c

    pltpu.sync_copy(zeros_hbm.at[pl.ds(s * ROWS_PER_TILE, ROWS_PER_TILE)],
                    acc_sh.at[pl.ds(s * ROWS_PER_TILE, ROWS_PER_TILE)])

    pltpu.sync_copy(edge_hbm.at[0, pl.ds(wid * ept, ept)], sidx_v)
    pltpu.sync_copy(edge_hbm.at[1, pl.ds(wid * ept, ept)], didx_v)
    pltpu.sync_copy(g_hbm.at[pl.ds(s * ROWS_PER_TILE, ROWS_PER_TILE)],
                    gtab_sh.at[pl.ds(s * ROWS_PER_TILE, ROWS_PER_TILE)])

    plsc.subcore_barrier()

    def gather_start(g, p, k):
      pltpu.async_copy(gtab_sh.at[sidx_v.at[pl.ds((g * NBUF + k) * CH, CH)]],
                       rows_v.at[p, k], gsem)

    def gather_wait(g, p, k):
      pltpu.make_async_copy(
          gtab_sh.at[sidx_v.at[pl.ds((g * NBUF + k) * CH, CH)]],
          rows_v.at[p, k], gsem).wait()

    def scatter_start(g, p, k):
      pltpu.async_copy(rows_v.at[p, k],
                       acc_sh.at[didx_v.at[pl.ds((g * NBUF + k) * CH, CH)]],
                       ssem, add=True)

    def scatter_wait(g, p, k):
      pltpu.make_async_copy(
          rows_v.at[p, k],
          acc_sh.at[didx_v.at[pl.ds((g * NBUF + k) * CH, CH)]], ssem).wait()

    for k in range(NBUF):
      gather_start(0, 0, k)

    def group_body(g, carry):
      p = lax.rem(g, 2)
      for k in range(NBUF):
        gather_wait(g, p, k)

      @pl.when(g < n_groups - 1)
      def _():
        for k in range(NBUF):
          gather_start(g + 1, 1 - p, k)

      for k in range(NBUF):
        scatter_start(g, p, k)
      for k in range(NBUF):
        scatter_wait(g, p, k)
      return carry

    lax.fori_loop(0, n_groups, group_body, 0)

    plsc.subcore_barrier()

    pltpu.sync_copy(acc_sh.at[pl.ds(s * ROWS_PER_TILE, ROWS_PER_TILE)],
                    out_hbm.at[c, pl.ds(s * ROWS_PER_TILE, ROWS_PER_TILE)])

  return prop_kernel


def _mm_body(cnt_ref, x_ref, w_ref, g_ref, dinv_ref):
  deg = jnp.sum(cnt_ref[...], axis=0) + 1.0
  dinv = lax.rsqrt(deg)
  h = jnp.dot(x_ref[...], w_ref[...], preferred_element_type=jnp.float32)
  g_ref[...] = h * dinv[:, None]
  dinv_ref[...] = dinv[None, :]


def _fin_body(s_ref, g_ref, dinv_ref, b_ref, o_ref):
  t = s_ref[0] + s_ref[1] + g_ref[...]
  o_ref[...] = t * dinv_ref[0][:, None] + b_ref[...]


def kernel(x, edge_index, W, b):
  n = x.shape[0]
  e = edge_index.shape[1]
  d_out = W.shape[1]

  ei = edge_index.astype(jnp.int32)
  ept = e // NTILES

  xp = jnp.pad(x, ((0, NP - n), (0, 0)))
  Wp = jnp.pad(W, ((0, 0), (0, DP - d_out)))
  b2 = jnp.pad(b, (0, DP - d_out))[None, :]

  cnt = _deg_kernel_factory(ept)(ei)

  blk = 1024
  g, dinv = pl.pallas_call(
      _mm_body,
      grid=(NP // blk,),
      in_specs=[
          pl.BlockSpec((NTILES, blk), lambda i: (0, i)),
          pl.BlockSpec((blk, 128), lambda i: (i, 0)),
          pl.BlockSpec((128, DP), lambda i: (0, 0)),
      ],
      out_specs=[
          pl.BlockSpec((blk, DP), lambda i: (i, 0)),
          pl.BlockSpec((1, blk), lambda i: (0, i)),
      ],
      out_shape=[
          jax.ShapeDtypeStruct((NP, DP), jnp.float32),
          jax.ShapeDtypeStruct((1, NP), jnp.float32),
      ],
  )(cnt, xp, Wp)

  zeros = jnp.zeros((NP, DP), jnp.float32)
  S = _prop_kernel_factory(ept)(g, ei, zeros)

  outp = pl.pallas_call(
      _fin_body,
      grid=(NP // blk,),
      in_specs=[
          pl.BlockSpec((2, blk, DP), lambda i: (0, i, 0)),
          pl.BlockSpec((blk, DP), lambda i: (i, 0)),
          pl.BlockSpec((1, blk), lambda i: (0, i)),
          pl.BlockSpec((1, DP), lambda i: (0, 0)),
      ],
      out_specs=pl.BlockSpec((blk, DP), lambda i: (i, 0)),
      out_shape=jax.ShapeDtypeStruct((NP, DP), jnp.float32),
  )(S, g, dinv, b2)

  return outp[:n, :d_out]


# speedup vs baseline: 62.5447x; 1.0164x over previous
import functools

import jax
import jax.numpy as jnp
from jax import lax
from jax.experimental import pallas as pl
from jax.experimental.pallas import tpu as pltpu
from jax.experimental.pallas import tpu_sc as plsc

NP = 10240
DP = 40
NTILES = 32
ROWS_PER_TILE = NP // 16

_mesh = plsc.VectorSubcoreMesh(core_axis_name="c", subcore_axis_name="s")


def _deg_kernel_factory(ept):
  @functools.partial(
      pl.kernel,
      out_type=jax.ShapeDtypeStruct((NTILES, NP), jnp.float32),
      mesh=_mesh,
      scratch_types=[
          pltpu.VMEM((ept,), jnp.int32),
          pltpu.VMEM((NP,), jnp.float32),
      ],
      compiler_params=pltpu.CompilerParams(needs_layout_passes=False,
                                           use_tc_tiling_on_sc=False),
  )
  def deg_kernel(edge_hbm, out_hbm, idx_v, deg_v):
    c = lax.axis_index("c")
    s = lax.axis_index("s")
    wid = s * 2 + c

    def zero_body(i, carry):
      for u in range(8):
        deg_v[pl.ds(i * 128 + u * 16, 16)] = jnp.zeros((16,), jnp.float32)
      return carry

    lax.fori_loop(0, NP // 128, zero_body, 0)

    pltpu.sync_copy(edge_hbm.at[1, pl.ds(wid * ept, ept)], idx_v)

    ones = jnp.ones((16,), jnp.float32)

    def body(i, carry):
      for u in range(8):
        idx = idx_v[pl.ds(i * 128 + u * 16, 16)]
        plsc.addupdate_scatter(deg_v, [idx], ones)
      return carry

    lax.fori_loop(0, ept // 128, body, 0)
    for u in range((ept % 128) // 16):
      idx = idx_v[pl.ds((ept // 128) * 128 + u * 16, 16)]
      plsc.addupdate_scatter(deg_v, [idx], ones)

    pltpu.sync_copy(deg_v, out_hbm.at[wid])

  return deg_kernel


NBUF = 2
CH = 200


def _prop_kernel_factory(ept):
  n_chunks = ept // CH
  n_groups = n_chunks // NBUF

  @functools.partial(
      pl.kernel,
      out_type=jax.ShapeDtypeStruct((2, NP, DP), jnp.float32),
      mesh=_mesh,
      scratch_types=[
          pltpu.VMEM((ept,), jnp.int32),
          pltpu.VMEM((ept,), jnp.int32),
          pltpu.VMEM((2, NBUF, CH, DP), jnp.float32),
          pltpu.VMEM_SHARED((NP, DP), jnp.float32),
          pltpu.VMEM_SHARED((NP, DP), jnp.float32),
          pltpu.SemaphoreType.DMA,
          pltpu.SemaphoreType.DMA,
      ],
      compiler_params=pltpu.CompilerParams(use_tc_tiling_on_sc=False),
  )
  def prop_kernel(g_hbm, edge_hbm, zeros_hbm, out_hbm, sidx_v, didx_v,
                  rows_v, acc_sh, gtab_sh, gsem, ssem):
    c = lax.axis_index("c")
    s = lax.axis_index("s")
    wid = s * 2 + c

    @pl.when(c == 0)
    def _():
      pltpu.sync_copy(g_hbm.at[pl.ds(s * ROWS_PER_TILE, ROWS_PER_TILE)],
                      acc_sh.at[pl.ds(s * ROWS_PER_TILE, ROWS_PER_TILE)])

    @pl.when(c == 1)
    def _():
      pltpu.sync_copy(zeros_hbm.at[pl.ds(s * ROWS_PER_TILE, ROWS_PER_TILE)],
                      acc_sh.at[pl.ds(s * ROWS_PER_TILE, ROWS_PER_TILE)])

    pltpu.sync_copy(edge_hbm.at[0, pl.ds(wid * ept, ept)], sidx_v)
    pltpu.sync_copy(edge_hbm.at[1, pl.ds(wid * ept, ept)], didx_v)
    pltpu.sync_copy(g_hbm.at[pl.ds(s * ROWS_PER_TILE, ROWS_PER_TILE)],
                    gtab_sh.at[pl.ds(s * ROWS_PER_TILE, ROWS_PER_TILE)])

    plsc.subcore_barrier()

    def gather_start(g, p, k):
      pltpu.async_copy(gtab_sh.at[sidx_v.at[pl.ds((g * NBUF + k) * CH, CH)]],
                       rows_v.at[p, k], gsem)

    def gather_wait(g, p, k):
      pltpu.make_async_copy(
          gtab_sh.at[sidx_v.at[pl.ds((g * NBUF + k) * CH, CH)]],
          rows_v.at[p, k], gsem).wait()

    def scatter_start(g, p, k):
      pltpu.async_copy(rows_v.at[p, k],
                       acc_sh.at[didx_v.at[pl.ds((g * NBUF + k) * CH, CH)]],
                       ssem, add=True)

    def scatter_wait(g, p, k):
      pltpu.make_async_copy(
          rows_v.at[p, k],
          acc_sh.at[didx_v.at[pl.ds((g * NBUF + k) * CH, CH)]], ssem).wait()

    for k in range(NBUF):
      gather_start(0, 0, k)

    def group_body(g, carry):
      p = lax.rem(g, 2)
      for k in range(NBUF):
        gather_wait(g, p, k)

      @pl.when(g < n_groups - 1)
      def _():
        for k in range(NBUF):
          gather_start(g + 1, 1 - p, k)

      for k in range(NBUF):
        scatter_start(g, p, k)
      for k in range(NBUF):
        scatter_wait(g, p, k)
      return carry

    lax.fori_loop(0, n_groups, group_body, 0)

    plsc.subcore_barrier()

    pltpu.sync_copy(acc_sh.at[pl.ds(s * ROWS_PER_TILE, ROWS_PER_TILE)],
                    out_hbm.at[c, pl.ds(s * ROWS_PER_TILE, ROWS_PER_TILE)])

  return prop_kernel


def _mm_body(cnt_ref, x_ref, w_ref, g_ref, dinv_ref):
  deg = jnp.sum(cnt_ref[...], axis=0) + 1.0
  dinv = lax.rsqrt(deg)
  h = jnp.dot(x_ref[...], w_ref[...], preferred_element_type=jnp.float32)
  g_ref[...] = h * dinv[:, None]
  dinv_ref[...] = dinv[None, :]


def _fin_body(s_ref, dinv_ref, b_ref, o_ref):
  t = s_ref[0] + s_ref[1]
  o_ref[...] = t * dinv_ref[0][:, None] + b_ref[...]


def kernel(x, edge_index, W, b):
  n = x.shape[0]
  e = edge_index.shape[1]
  d_out = W.shape[1]

  ei = edge_index.astype(jnp.int32)
  ept = e // NTILES

  Wp = jnp.pad(W, ((0, 0), (0, DP - d_out)))
  b2 = jnp.pad(b, (0, DP - d_out))[None, :]

  cnt = _deg_kernel_factory(ept)(ei)

  blk = 1024
  g, dinv = pl.pallas_call(
      _mm_body,
      grid=(NP // blk,),
      in_specs=[
          pl.BlockSpec((NTILES, blk), lambda i: (0, i)),
          pl.BlockSpec((blk, 128), lambda i: (i, 0)),
          pl.BlockSpec((128, DP), lambda i: (0, 0)),
      ],
      out_specs=[
          pl.BlockSpec((blk, DP), lambda i: (i, 0)),
          pl.BlockSpec((1, blk), lambda i: (0, i)),
      ],
      out_shape=[
          jax.ShapeDtypeStruct((NP, DP), jnp.float32),
          jax.ShapeDtypeStruct((1, NP), jnp.float32),
      ],
  )(cnt, x, Wp)

  zeros = jnp.zeros((NP, DP), jnp.float32)
  S = _prop_kernel_factory(ept)(g, ei, zeros)

  outp = pl.pallas_call(
      _fin_body,
      grid=(NP // blk,),
      in_specs=[
          pl.BlockSpec((2, blk, DP), lambda i: (0, i, 0)),
          pl.BlockSpec((1, blk), lambda i: (0, i)),
          pl.BlockSpec((1, DP), lambda i: (0, 0)),
      ],
      out_specs=pl.BlockSpec((blk, DP), lambda i: (i, 0)),
      out_shape=jax.ShapeDtypeStruct((n, DP), jnp.float32),
  )(S, dinv, b2)

  return outp[:, :d_out]
